# Initial kernel scaffold; baseline (speedup 1.0000x reference)
import jax
import jax.numpy as jnp
from jax.experimental import pallas as pl


def kernel(q, k):
    raise NotImplementedError("write your pallas kernel here")


# speedup vs baseline: 1.0274x; 1.0274x over previous
import jax
import jax.numpy as jnp
from jax.experimental import pallas as pl
from jax.experimental.pallas import tpu as pltpu

B = 8192
D = 256
TEMP = 0.05
INV_TEMP = 1.0 / TEMP
EPS = 1e-5
NORM_EPS = 1e-12

BMP = 512
NBP = B // BMP
BM = 1024
BN = 1024
NI = B // BM
NJ = B // BN


def _prep_kernel(q_ref, k_ref, qn_ref, kn_ref, d_ref):
    q = q_ref[...]
    k = k_ref[...]
    qs = jnp.sum(q * q, axis=1, keepdims=True)
    ks = jnp.sum(k * k, axis=1, keepdims=True)
    qn = q * (1.0 / jnp.maximum(jnp.sqrt(qs), NORM_EPS))
    kn = k * (1.0 / jnp.maximum(jnp.sqrt(ks), NORM_EPS))
    qn_ref[...] = qn.astype(jnp.bfloat16)
    kn_ref[...] = kn.astype(jnp.bfloat16)
    d_ref[...] = jnp.sum(qn * kn, axis=1)[None, None, :]


def _main_kernel(qn_ref, kn_ref, row_ref, colp_ref):
    j = pl.program_id(1)
    s = jax.lax.dot_general(
        qn_ref[...], kn_ref[...],
        (((1,), (1,)), ((), ())),
        preferred_element_type=jnp.float32,
    )
    e = jnp.exp(s * INV_TEMP)
    rs = jnp.sum(e, axis=1)[None, None, :]
    colp_ref[...] = jnp.sum(e, axis=0)[None, None, :]

    @pl.when(j == 0)
    def _():
        row_ref[...] = rs

    @pl.when(j != 0)
    def _():
        row_ref[...] = row_ref[...] + rs


def _final_kernel(row_ref, colp_ref, d_ref, o_ref):
    d = jnp.exp(d_ref[...] * INV_TEMP)
    den_qk = row_ref[...]
    den_kq = jnp.sum(colp_ref[...], axis=0, keepdims=True)
    lq = -jnp.log(d / den_qk + EPS)
    lk = -jnp.log(d / den_kq + EPS)
    o_ref[...] = jnp.reshape((jnp.sum(lq) + jnp.sum(lk)) * (1.0 / B), (1, 1))


def kernel(q, k):
    qn, kn, d3 = pl.pallas_call(
        _prep_kernel,
        grid=(NBP,),
        in_specs=[
            pl.BlockSpec((BMP, D), lambda i: (i, 0)),
            pl.BlockSpec((BMP, D), lambda i: (i, 0)),
        ],
        out_specs=[
            pl.BlockSpec((BMP, D), lambda i: (i, 0)),
            pl.BlockSpec((BMP, D), lambda i: (i, 0)),
            pl.BlockSpec((1, 1, BMP), lambda i: (i, 0, 0)),
        ],
        out_shape=[
            jax.ShapeDtypeStruct((B, D), jnp.bfloat16),
            jax.ShapeDtypeStruct((B, D), jnp.bfloat16),
            jax.ShapeDtypeStruct((NBP, 1, BMP), jnp.float32),
        ],
        compiler_params=pltpu.CompilerParams(
            dimension_semantics=("parallel",),
        ),
    )(q, k)

    row3, colp3 = pl.pallas_call(
        _main_kernel,
        grid=(NI, NJ),
        in_specs=[
            pl.BlockSpec((BM, D), lambda i, j: (i, 0)),
            pl.BlockSpec((BN, D), lambda i, j: (j, 0)),
        ],
        out_specs=[
            pl.BlockSpec((1, 1, BM), lambda i, j: (i, 0, 0)),
            pl.BlockSpec((1, 1, BN), lambda i, j: (i, 0, j)),
        ],
        out_shape=[
            jax.ShapeDtypeStruct((NI, 1, BM), jnp.float32),
            jax.ShapeDtypeStruct((NI, 1, B), jnp.float32),
        ],
        compiler_params=pltpu.CompilerParams(
            dimension_semantics=("parallel", "arbitrary"),
        ),
    )(qn, kn)

    loss = pl.pallas_call(
        _final_kernel,
        in_specs=[
            pl.BlockSpec((1, B), lambda: (0, 0)),
            pl.BlockSpec((NI, B), lambda: (0, 0)),
            pl.BlockSpec((1, B), lambda: (0, 0)),
        ],
        out_specs=pl.BlockSpec((1, 1), lambda: (0, 0)),
        out_shape=jax.ShapeDtypeStruct((1, 1), jnp.float32),
    )(row3.reshape(1, B), colp3.reshape(NI, B), d3.reshape(1, B))

    return jnp.reshape(loss, ())


# speedup vs baseline: 1.3286x; 1.2933x over previous
import jax
import jax.numpy as jnp
from jax.experimental import pallas as pl
from jax.experimental.pallas import tpu as pltpu

B = 8192
D = 256
TEMP = 0.05
EPS = 1e-5
NORM_EPS = 1e-12
LOG2E = 1.4426950408889634
SC = LOG2E / TEMP

BMP = 512
NBP = B // BMP
BM = 1024
BN = 1024
NI = B // BM
NJ = B // BN
LN = 128


def _prep_kernel(q_ref, k_ref, qn_ref, kn_ref, d_ref, dp_ref):
    q = q_ref[...]
    k = k_ref[...]
    qs = jnp.sum(q * q, axis=1, keepdims=True)
    ks = jnp.sum(k * k, axis=1, keepdims=True)
    qn = q * (1.0 / jnp.maximum(jnp.sqrt(qs), NORM_EPS))
    kn = k * (1.0 / jnp.maximum(jnp.sqrt(ks), NORM_EPS))
    qk = qn * kn
    d_ref[...] = jnp.sum(qk, axis=1)[None, None, :]
    dp_ref[...] = qk[:, :LN] + qk[:, LN:]
    qn_ref[...] = (qn * SC).astype(jnp.bfloat16)
    kn_ref[...] = kn.astype(jnp.bfloat16)


def _main_kernel(qn_ref, kn_ref, dp_ref, colp_ref, rloss_ref, racc_ref):
    j = pl.program_id(1)
    s = jax.lax.dot_general(
        qn_ref[...], kn_ref[...],
        (((1,), (1,)), ((), ())),
        preferred_element_type=jnp.float32,
    )
    e = jnp.exp2(s)
    rs = e[:, 0:LN]
    for c in range(1, BN // LN):
        rs = rs + e[:, c * LN:(c + 1) * LN]
    colp_ref[...] = jnp.sum(e, axis=0)[None, None, :]

    @pl.when(j == 0)
    def _():
        racc_ref[...] = rs

    @pl.when(j != 0)
    def _():
        racc_ref[...] = racc_ref[...] + rs

    @pl.when(j == NJ - 1)
    def _():
        ones = jnp.ones((LN, LN), jnp.bfloat16)
        den = jax.lax.dot_general(
            racc_ref[...].astype(jnp.bfloat16), ones,
            (((1,), (0,)), ((), ())),
            preferred_element_type=jnp.float32,
        )
        drep = jax.lax.dot_general(
            dp_ref[...].astype(jnp.bfloat16), ones,
            (((1,), (0,)), ((), ())),
            preferred_element_type=jnp.float32,
        )
        dexp = jnp.exp2(drep * SC)
        lq = -jnp.log(dexp / den + EPS)
        rloss_ref[...] = jnp.sum(lq, axis=0)[None, None, :]


def _final_kernel(rloss_ref, colp_ref, d_ref, o_ref):
    dexp = jnp.exp2(d_ref[...] * SC)
    den_kq = jnp.sum(colp_ref[...], axis=0, keepdims=True)
    lk = -jnp.log(dexp / den_kq + EPS)
    rl = jnp.sum(rloss_ref[...]) * (1.0 / LN)
    o_ref[...] = jnp.reshape((rl + jnp.sum(lk)) * (1.0 / B), (1, 1))


def kernel(q, k):
    qn, kn, d3, dp = pl.pallas_call(
        _prep_kernel,
        grid=(NBP,),
        in_specs=[
            pl.BlockSpec((BMP, D), lambda i: (i, 0)),
            pl.BlockSpec((BMP, D), lambda i: (i, 0)),
        ],
        out_specs=[
            pl.BlockSpec((BMP, D), lambda i: (i, 0)),
            pl.BlockSpec((BMP, D), lambda i: (i, 0)),
            pl.BlockSpec((1, 1, BMP), lambda i: (i, 0, 0)),
            pl.BlockSpec((BMP, LN), lambda i: (i, 0)),
        ],
        out_shape=[
            jax.ShapeDtypeStruct((B, D), jnp.bfloat16),
            jax.ShapeDtypeStruct((B, D), jnp.bfloat16),
            jax.ShapeDtypeStruct((NBP, 1, BMP), jnp.float32),
            jax.ShapeDtypeStruct((B, LN), jnp.float32),
        ],
        compiler_params=pltpu.CompilerParams(
            dimension_semantics=("parallel",),
        ),
    )(q, k)

    colp3, rloss3 = pl.pallas_call(
        _main_kernel,
        grid=(NI, NJ),
        in_specs=[
            pl.BlockSpec((BM, D), lambda i, j: (i, 0)),
            pl.BlockSpec((BN, D), lambda i, j: (j, 0)),
            pl.BlockSpec((BM, LN), lambda i, j: (i, 0)),
        ],
        out_specs=[
            pl.BlockSpec((1, 1, BN), lambda i, j: (i, 0, j)),
            pl.BlockSpec((1, 1, LN), lambda i, j: (i, 0, 0)),
        ],
        out_shape=[
            jax.ShapeDtypeStruct((NI, 1, B), jnp.float32),
            jax.ShapeDtypeStruct((NI, 1, LN), jnp.float32),
        ],
        scratch_shapes=[pltpu.VMEM((BM, LN), jnp.float32)],
        compiler_params=pltpu.CompilerParams(
            dimension_semantics=("parallel", "arbitrary"),
        ),
    )(qn, kn, dp)

    loss = pl.pallas_call(
        _final_kernel,
        in_specs=[
            pl.BlockSpec((NI, LN), lambda: (0, 0)),
            pl.BlockSpec((NI, B), lambda: (0, 0)),
            pl.BlockSpec((1, B), lambda: (0, 0)),
        ],
        out_specs=pl.BlockSpec((1, 1), lambda: (0, 0)),
        out_shape=jax.ShapeDtypeStruct((1, 1), jnp.float32),
    )(rloss3.reshape(NI, LN), colp3.reshape(NI, B), d3.reshape(1, B))

    return jnp.reshape(loss, ())


# speedup vs baseline: 1.8315x; 1.3785x over previous
import jax
import jax.numpy as jnp
from jax.experimental import pallas as pl
from jax.experimental.pallas import tpu as pltpu

B = 8192
D = 256
TEMP = 0.05
EPS = 1e-5
NORM_EPS = 1e-12
LOG2E = 1.4426950408889634
SC = LOG2E / TEMP

BMP = 512
NBP = B // BMP
BM = 1024
BN = 1024
NI = B // BM
NJ = B // BN
LN = 128


def _prep_kernel(q_ref, k_ref, qn_ref, kn_ref, d_ref, dp_ref):
    q = q_ref[...]
    k = k_ref[...]
    qs = jnp.sum(q * q, axis=1, keepdims=True)
    ks = jnp.sum(k * k, axis=1, keepdims=True)
    qn = q * (1.0 / jnp.maximum(jnp.sqrt(qs), NORM_EPS))
    kn = k * (1.0 / jnp.maximum(jnp.sqrt(ks), NORM_EPS))
    qk = qn * kn
    d_ref[...] = jnp.sum(qk, axis=1)[None, None, :]
    dp_ref[...] = (qk[:, :LN] + qk[:, LN:]).astype(jnp.bfloat16)
    qn_ref[...] = (qn * SC).astype(jnp.bfloat16)
    kn_ref[...] = kn.astype(jnp.bfloat16)


def _main_kernel(qn_ref, kn_ref, dp_ref, colp_ref, rloss_ref):
    qb = qn_ref[...]
    rs = None
    for c in range(NJ):
        kb = kn_ref[c * BN:(c + 1) * BN, :]
        s = jax.lax.dot_general(
            qb, kb,
            (((1,), (1,)), ((), ())),
            preferred_element_type=jnp.float32,
        )
        e = jnp.exp2(s)
        acc = e[:, 0:LN]
        for cc in range(1, BN // LN):
            acc = acc + e[:, cc * LN:(cc + 1) * LN]
        rs = acc if rs is None else rs + acc
        colp_ref[:, :, c * BN:(c + 1) * BN] = jnp.sum(e, axis=0)[None, None, :]

    ones = jnp.ones((LN, LN), jnp.bfloat16)
    den = jax.lax.dot_general(
        rs.astype(jnp.bfloat16), ones,
        (((1,), (0,)), ((), ())),
        preferred_element_type=jnp.float32,
    )
    drep = jax.lax.dot_general(
        dp_ref[...], ones,
        (((1,), (0,)), ((), ())),
        preferred_element_type=jnp.float32,
    )
    dexp = jnp.exp2(drep * SC)
    lq = -jnp.log(dexp / den + EPS)
    rloss_ref[...] = jnp.sum(lq, axis=0)[None, None, :]


def _final_kernel(rloss_ref, colp_ref, d_ref, o_ref):
    dexp = jnp.exp2(d_ref[...] * SC)
    den_kq = jnp.sum(colp_ref[...], axis=0, keepdims=True)
    lk = -jnp.log(dexp / den_kq + EPS)
    rl = jnp.sum(rloss_ref[...]) * (1.0 / LN)
    o_ref[...] = jnp.reshape((rl + jnp.sum(lk)) * (1.0 / B), (1, 1))


def kernel(q, k):
    qn, kn, d3, dp = pl.pallas_call(
        _prep_kernel,
        grid=(NBP,),
        in_specs=[
            pl.BlockSpec((BMP, D), lambda i: (i, 0)),
            pl.BlockSpec((BMP, D), lambda i: (i, 0)),
        ],
        out_specs=[
            pl.BlockSpec((BMP, D), lambda i: (i, 0)),
            pl.BlockSpec((BMP, D), lambda i: (i, 0)),
            pl.BlockSpec((1, 1, BMP), lambda i: (i, 0, 0)),
            pl.BlockSpec((BMP, LN), lambda i: (i, 0)),
        ],
        out_shape=[
            jax.ShapeDtypeStruct((B, D), jnp.bfloat16),
            jax.ShapeDtypeStruct((B, D), jnp.bfloat16),
            jax.ShapeDtypeStruct((NBP, 1, BMP), jnp.float32),
            jax.ShapeDtypeStruct((B, LN), jnp.bfloat16),
        ],
        compiler_params=pltpu.CompilerParams(
            dimension_semantics=("parallel",),
        ),
    )(q, k)

    colp3, rloss3 = pl.pallas_call(
        _main_kernel,
        grid=(NI,),
        in_specs=[
            pl.BlockSpec((BM, D), lambda i: (i, 0)),
            pl.BlockSpec((B, D), lambda i: (0, 0)),
            pl.BlockSpec((BM, LN), lambda i: (i, 0)),
        ],
        out_specs=[
            pl.BlockSpec((1, 1, B), lambda i: (i, 0, 0)),
            pl.BlockSpec((1, 1, LN), lambda i: (i, 0, 0)),
        ],
        out_shape=[
            jax.ShapeDtypeStruct((NI, 1, B), jnp.float32),
            jax.ShapeDtypeStruct((NI, 1, LN), jnp.float32),
        ],
        compiler_params=pltpu.CompilerParams(
            dimension_semantics=("parallel",),
            vmem_limit_bytes=40 * 1024 * 1024,
        ),
    )(qn, kn, dp)

    loss = pl.pallas_call(
        _final_kernel,
        in_specs=[
            pl.BlockSpec((NI, LN), lambda: (0, 0)),
            pl.BlockSpec((NI, B), lambda: (0, 0)),
            pl.BlockSpec((1, B), lambda: (0, 0)),
        ],
        out_specs=pl.BlockSpec((1, 1), lambda: (0, 0)),
        out_shape=jax.ShapeDtypeStruct((1, 1), jnp.float32),
    )(rloss3.reshape(NI, LN), colp3.reshape(NI, B), d3.reshape(1, B))

    return jnp.reshape(loss, ())


# speedup vs baseline: 2.0517x; 1.1203x over previous
import jax
import jax.numpy as jnp
from jax.experimental import pallas as pl
from jax.experimental.pallas import tpu as pltpu

B = 8192
D = 256
TEMP = 0.05
EPS = 1e-5
NORM_EPS = 1e-12
LOG2E = 1.4426950408889634
SC = LOG2E / TEMP
SQ = SC ** 0.5

BMP = 1024
NBP = B // BMP
BM = 1024
BN = 1024
NI = B // BM
NJ = B // BN
LN = 128


def _prep_kernel(q_ref, k_ref, qn_ref, kn_ref, d_ref, dp_ref):
    q = q_ref[...]
    k = k_ref[...]
    qs = jnp.sum(q * q, axis=1, keepdims=True)
    ks = jnp.sum(k * k, axis=1, keepdims=True)
    qn = q * (1.0 / jnp.maximum(jnp.sqrt(qs), NORM_EPS))
    kn = k * (1.0 / jnp.maximum(jnp.sqrt(ks), NORM_EPS))
    qk = qn * kn
    d_ref[...] = jnp.sum(qk, axis=1)[None, :]
    dp_ref[...] = (qk[:, :LN] + qk[:, LN:]).astype(jnp.bfloat16)
    qn_ref[...] = (qn * SQ).astype(jnp.float8_e4m3fn)
    kn_ref[...] = (kn * SQ).astype(jnp.float8_e4m3fn)


def _main_kernel(qn_ref, kn_ref, dp_ref, colp_ref, rloss_ref):
    qb = qn_ref[...]
    rs = None
    for c in range(NJ):
        kb = kn_ref[c * BN:(c + 1) * BN, :]
        s = jax.lax.dot_general(
            qb, kb,
            (((1,), (1,)), ((), ())),
            preferred_element_type=jnp.float32,
        )
        e = jnp.exp2(s)
        acc = e[:, 0:LN]
        for cc in range(1, BN // LN):
            acc = acc + e[:, cc * LN:(cc + 1) * LN]
        rs = acc if rs is None else rs + acc
        colp_ref[:, :, c * BN:(c + 1) * BN] = jnp.sum(e, axis=0)[None, None, :]

    ones = jnp.ones((LN, LN), jnp.bfloat16)
    den = jax.lax.dot_general(
        rs.astype(jnp.bfloat16), ones,
        (((1,), (0,)), ((), ())),
        preferred_element_type=jnp.float32,
    )
    drep = jax.lax.dot_general(
        dp_ref[...], ones,
        (((1,), (0,)), ((), ())),
        preferred_element_type=jnp.float32,
    )
    dexp = jnp.exp2(drep * SC)
    lq = -jnp.log(dexp / den + EPS)
    rloss_ref[...] = jnp.sum(lq, axis=0)[None, None, :]


def _final_kernel(rloss_ref, colp_ref, d_ref, o_ref):
    dexp = jnp.exp2(d_ref[...] * SC)
    den_kq = jnp.sum(colp_ref[...], axis=0, keepdims=True)
    lk = -jnp.log(dexp / den_kq + EPS)
    rl = jnp.sum(rloss_ref[...]) * (1.0 / LN)
    o_ref[...] = jnp.reshape((rl + jnp.sum(lk)) * (1.0 / B), (1, 1))


def kernel(q, k):
    qn, kn, d3, dp = pl.pallas_call(
        _prep_kernel,
        grid=(NBP,),
        in_specs=[
            pl.BlockSpec((BMP, D), lambda i: (i, 0)),
            pl.BlockSpec((BMP, D), lambda i: (i, 0)),
        ],
        out_specs=[
            pl.BlockSpec((BMP, D), lambda i: (i, 0)),
            pl.BlockSpec((BMP, D), lambda i: (i, 0)),
            pl.BlockSpec((1, BMP), lambda i: (0, i)),
            pl.BlockSpec((BMP, LN), lambda i: (i, 0)),
        ],
        out_shape=[
            jax.ShapeDtypeStruct((B, D), jnp.float8_e4m3fn),
            jax.ShapeDtypeStruct((B, D), jnp.float8_e4m3fn),
            jax.ShapeDtypeStruct((1, B), jnp.float32),
            jax.ShapeDtypeStruct((B, LN), jnp.bfloat16),
        ],
        compiler_params=pltpu.CompilerParams(
            dimension_semantics=("parallel",),
        ),
    )(q, k)

    colp3, rloss3 = pl.pallas_call(
        _main_kernel,
        grid=(NI,),
        in_specs=[
            pl.BlockSpec((BM, D), lambda i: (i, 0)),
            pl.BlockSpec((B, D), lambda i: (0, 0)),
            pl.BlockSpec((BM, LN), lambda i: (i, 0)),
        ],
        out_specs=[
            pl.BlockSpec((1, 1, B), lambda i: (i, 0, 0)),
            pl.BlockSpec((1, 1, LN), lambda i: (i, 0, 0)),
        ],
        out_shape=[
            jax.ShapeDtypeStruct((NI, 1, B), jnp.float32),
            jax.ShapeDtypeStruct((NI, 1, LN), jnp.float32),
        ],
        compiler_params=pltpu.CompilerParams(
            dimension_semantics=("parallel",),
            vmem_limit_bytes=40 * 1024 * 1024,
        ),
    )(qn, kn, dp)

    loss = pl.pallas_call(
        _final_kernel,
        in_specs=[
            pl.BlockSpec((NI, LN), lambda: (0, 0)),
            pl.BlockSpec((NI, B), lambda: (0, 0)),
            pl.BlockSpec((1, B), lambda: (0, 0)),
        ],
        out_specs=pl.BlockSpec((1, 1), lambda: (0, 0)),
        out_shape=jax.ShapeDtypeStruct((1, 1), jnp.float32),
    )(rloss3.reshape(NI, LN), colp3.reshape(NI, B), d3)

    return jnp.reshape(loss, ())


# speedup vs baseline: 2.1930x; 1.0688x over previous
import jax
import jax.numpy as jnp
from jax.experimental import pallas as pl
from jax.experimental.pallas import tpu as pltpu

B = 8192
D = 256
TEMP = 0.05
EPS = 1e-5
NORM_EPS = 1e-12
LOG2E = 1.4426950408889634
SC = LOG2E / TEMP
SQ = SC ** 0.5

BM = 1024
BN = 1024
NI = B // BM
NJ = B // BN
LN = 128


def _kprep_kernel(k_ref, kn8_ref):
    k = k_ref[...]
    ks = jnp.sum(k * k, axis=1, keepdims=True)
    kn = k * (1.0 / jnp.maximum(jnp.sqrt(ks), NORM_EPS))
    kn8_ref[...] = (kn * SQ).astype(jnp.float8_e4m3fn)


def _main_kernel(q_ref, kn8_ref, kb8_ref, colp_ref, rloss_ref, d_ref):
    q = q_ref[...]
    qs = jnp.sum(q * q, axis=1, keepdims=True)
    qn = q * (1.0 / jnp.maximum(jnp.sqrt(qs), NORM_EPS))
    qb8 = (qn * SQ).astype(jnp.float8_e4m3fn)
    qk = qn * kb8_ref[...].astype(jnp.float32)
    dp = qk[:, :LN] + qk[:, LN:]
    dp_bf = dp.astype(jnp.bfloat16)

    rs = None
    for c in range(NJ):
        kb = kn8_ref[c * BN:(c + 1) * BN, :]
        s = jax.lax.dot_general(
            qb8, kb,
            (((1,), (1,)), ((), ())),
            preferred_element_type=jnp.float32,
        )
        e = jnp.exp2(s)
        acc = e[:, 0:LN]
        for cc in range(1, BN // LN):
            acc = acc + e[:, cc * LN:(cc + 1) * LN]
        rs = acc if rs is None else rs + acc
        colp_ref[:, :, c * BN:(c + 1) * BN] = jnp.sum(e, axis=0)[None, None, :]

    ones = jnp.ones((LN, LN), jnp.bfloat16)
    den = jax.lax.dot_general(
        rs.astype(jnp.bfloat16), ones,
        (((1,), (0,)), ((), ())),
        preferred_element_type=jnp.float32,
    )
    drep = jax.lax.dot_general(
        dp_bf, ones,
        (((1,), (0,)), ((), ())),
        preferred_element_type=jnp.float32,
    )
    dexp = jnp.exp2(drep * SQ)
    lq = -jnp.log(dexp / den + EPS)
    rloss_ref[...] = jnp.sum(lq, axis=0)[None, None, :]

    ones_row = jnp.ones((1, LN), jnp.bfloat16)
    d_ref[...] = jax.lax.dot_general(
        ones_row, dp_bf,
        (((1,), (1,)), ((), ())),
        preferred_element_type=jnp.float32,
    )


def _final_kernel(rloss_ref, colp_ref, d_ref, o_ref):
    dexp = jnp.exp2(d_ref[...] * SQ)
    den_kq = jnp.sum(colp_ref[...], axis=0, keepdims=True)
    lk = -jnp.log(dexp / den_kq + EPS)
    rl = jnp.sum(rloss_ref[...]) * (1.0 / LN)
    o_ref[...] = jnp.reshape((rl + jnp.sum(lk)) * (1.0 / B), (1, 1))


def kernel(q, k):
    kn8 = pl.pallas_call(
        _kprep_kernel,
        grid=(NI,),
        in_specs=[pl.BlockSpec((BM, D), lambda i: (i, 0))],
        out_specs=pl.BlockSpec((BM, D), lambda i: (i, 0)),
        out_shape=jax.ShapeDtypeStruct((B, D), jnp.float8_e4m3fn),
        compiler_params=pltpu.CompilerParams(
            dimension_semantics=("parallel",),
        ),
    )(k)

    colp3, rloss3, d3 = pl.pallas_call(
        _main_kernel,
        grid=(NI,),
        in_specs=[
            pl.BlockSpec((BM, D), lambda i: (i, 0)),
            pl.BlockSpec((B, D), lambda i: (0, 0)),
            pl.BlockSpec((BM, D), lambda i: (i, 0)),
        ],
        out_specs=[
            pl.BlockSpec((1, 1, B), lambda i: (i, 0, 0)),
            pl.BlockSpec((1, 1, LN), lambda i: (i, 0, 0)),
            pl.BlockSpec((1, BM), lambda i: (0, i)),
        ],
        out_shape=[
            jax.ShapeDtypeStruct((NI, 1, B), jnp.float32),
            jax.ShapeDtypeStruct((NI, 1, LN), jnp.float32),
            jax.ShapeDtypeStruct((1, B), jnp.float32),
        ],
        compiler_params=pltpu.CompilerParams(
            dimension_semantics=("parallel",),
            vmem_limit_bytes=40 * 1024 * 1024,
        ),
    )(q, kn8, kn8)

    loss = pl.pallas_call(
        _final_kernel,
        in_specs=[
            pl.BlockSpec((NI, LN), lambda: (0, 0)),
            pl.BlockSpec((NI, B), lambda: (0, 0)),
            pl.BlockSpec((1, B), lambda: (0, 0)),
        ],
        out_specs=pl.BlockSpec((1, 1), lambda: (0, 0)),
        out_shape=jax.ShapeDtypeStruct((1, 1), jnp.float32),
    )(rloss3.reshape(NI, LN), colp3.reshape(NI, B), d3)

    return jnp.reshape(loss, ())
